# baseline (speedup 1.0000x reference)
import functools

import jax
import jax.numpy as jnp
from jax import lax
from jax.experimental import pallas as pl
from jax.experimental.pallas import tpu as pltpu
from jax.experimental.pallas import tpu_sc as plsc

N = 100000
DIN = 128
DH = 16
DOUT = 40
E = 3200000

NC = 2
NS = 16
NW = NC * NS
EPW = E // NW
CHUNK = 80
NCHUNKS = EPW // CHUNK

DGC = 125
DNG = NCHUNKS // DGC
PGC = 25
PNG = NCHUNKS // PGC
NBUF = 5
ZBLK = 2000
NBLK = N // ZBLK
NPK = N * DH // 128
PB = 2048
NDB = PB * 8
TGRID = (NPK + PB - 1) // PB

def _node_blocks(s):
    out = []
    for j in range((NBLK + NS - 1) // NS):
        out.append(s + j * NS)
    return out


def _deg_kernel(dst_hbm, ew_hbm, zrow_hbm, out0_hbm, out1_hbm, acc, dstb, ewb,
                buf, bufx, ssem, isems):
    c = lax.axis_index("c")
    sid = lax.axis_index("s")
    wid = sid * NC + c
    wbase = wid * NCHUNKS

    pltpu.sync_copy(zrow_hbm, buf)
    for b in _node_blocks(sid):
        @pl.when(b < NBLK)
        def _():
            pltpu.sync_copy(buf, acc.at[pl.ds(b * ZBLK, ZBLK)])
    plsc.subcore_barrier()

    pltpu.async_copy(dst_hbm.at[pl.ds(wbase, DGC)], dstb.at[0], isems[0])
    pltpu.async_copy(ew_hbm.at[pl.ds(wbase, DGC)], ewb.at[0], isems[0])

    def do_group(go2, par):
        g = go2 * 2 + par
        gbase = wbase + g * DGC

        @pl.when(g > 0)
        def _():
            @pl.loop(0, DGC)
            def _(_k):
                pltpu.make_async_copy(
                    ewb.at[1 - par, 0], acc.at[dstb.at[1 - par, 0]], ssem
                ).wait()

        pltpu.make_async_copy(dst_hbm.at[pl.ds(gbase, DGC)], dstb.at[par],
                              isems[par]).wait()
        pltpu.make_async_copy(ew_hbm.at[pl.ds(gbase, DGC)], ewb.at[par],
                              isems[par]).wait()

        @pl.when(g + 1 < DNG)
        def _():
            pltpu.async_copy(dst_hbm.at[pl.ds(gbase + DGC, DGC)],
                             dstb.at[1 - par], isems[1 - par])
            pltpu.async_copy(ew_hbm.at[pl.ds(gbase + DGC, DGC)],
                             ewb.at[1 - par], isems[1 - par])

        @pl.loop(0, DGC)
        def _(k):
            pltpu.async_copy(ewb.at[par, k], acc.at[dstb.at[par, k]], ssem,
                             add=True)

    @pl.loop(0, DNG // 2)
    def _(go2):
        do_group(go2, 0)
        do_group(go2, 1)

    @pl.loop(0, DGC)
    def _(_k):
        pltpu.make_async_copy(ewb.at[1, 0], acc.at[dstb.at[1, 0]], ssem).wait()

    plsc.subcore_barrier()
    for b in _node_blocks(sid):
        @pl.when(b < NBLK)
        def _():
            pltpu.sync_copy(acc.at[pl.ds(b * ZBLK, ZBLK)], buf)

            @pl.loop(0, ZBLK // 16)
            def _(g):
                d16 = buf[pl.ds(g * 16, 16)]
                for t in range(16):
                    bufx[g * 16 + t, :] = jnp.broadcast_to(d16[t], (DH,))

            @pl.when(c == 0)
            def _():
                pltpu.sync_copy(bufx, out0_hbm.at[pl.ds(b * ZBLK, ZBLK)])

            @pl.when(c == 1)
            def _():
                pltpu.sync_copy(bufx, out1_hbm.at[pl.ds(b * ZBLK, ZBLK)])


def _prop_kernel(hp_hbm, src_hbm, dst_hbm, ew_hbm, zblk_hbm, out0_hbm, out1_hbm,
                 acc, srcb, dstb, ewb, rows_g, rows_s, gsems, ssems, isems):
    c = lax.axis_index("c")
    sid = lax.axis_index("s")
    wid = sid * NC + c
    wbase = wid * NCHUNKS

    for b in _node_blocks(sid):
        @pl.when(b < NBLK)
        def _():
            pltpu.sync_copy(zblk_hbm, acc.at[pl.ds(b * ZBLK, ZBLK)])
    plsc.subcore_barrier()

    def fire_idx(gbase, par, sem):
        pltpu.async_copy(src_hbm.at[pl.ds(gbase, PGC)], srcb.at[par], sem)
        pltpu.async_copy(dst_hbm.at[pl.ds(gbase, PGC)], dstb.at[par], sem)
        pltpu.async_copy(ew_hbm.at[pl.ds(gbase, PGC)], ewb.at[par], sem)

    def wait_idx(gbase, par, sem):
        pltpu.make_async_copy(src_hbm.at[pl.ds(gbase, PGC)], srcb.at[par],
                              sem).wait()
        pltpu.make_async_copy(dst_hbm.at[pl.ds(gbase, PGC)], dstb.at[par],
                              sem).wait()
        pltpu.make_async_copy(ew_hbm.at[pl.ds(gbase, PGC)], ewb.at[par],
                              sem).wait()

    def scale(sl, par, k):
        for q in range(CHUNK // 16):
            ew16 = ewb[par, k, pl.ds(q * 16, 16)]
            for t in range(16):
                i = q * 16 + t
                rows_s[sl, i, :] = rows_g[sl, i, :] * ew16[t]

    HC = CHUNK // 2

    def fire_gather(par, row, sl):
        pltpu.async_copy(hp_hbm.at[srcb.at[par, row, pl.ds(0, HC)]],
                         rows_g.at[sl, pl.ds(0, HC)], gsems[sl])
        pltpu.async_copy(hp_hbm.at[srcb.at[par, row, pl.ds(HC, HC)]],
                         rows_g.at[sl, pl.ds(HC, HC)], gsems[sl])

    def wait_gather(sl):
        pltpu.make_async_copy(hp_hbm.at[srcb.at[0, 0, pl.ds(0, HC)]],
                              rows_g.at[sl, pl.ds(0, HC)], gsems[sl]).wait()
        pltpu.make_async_copy(hp_hbm.at[srcb.at[0, 0, pl.ds(HC, HC)]],
                              rows_g.at[sl, pl.ds(HC, HC)], gsems[sl]).wait()

    def fire_scatter(par, k, sl):
        pltpu.async_copy(rows_s.at[sl], acc.at[dstb.at[par, k]], ssems[sl],
                         add=True)

    def wait_scatter(sl):
        pltpu.make_async_copy(rows_s.at[sl], acc.at[dstb.at[0, 0]],
                              ssems[sl]).wait()

    fire_idx(wbase, 0, isems[0])
    wait_idx(wbase, 0, isems[0])
    for sl in range(NBUF):
        fire_gather(0, sl, sl)

    TRIPS = PGC // NBUF

    def do_group(go2, par):
        g = go2 * 2 + par
        gbase = wbase + g * PGC

        for sl in range(NBUF):
            wait_gather(sl)

            @pl.when(g > 0)
            def _():
                wait_scatter(sl)

            scale(sl, par, sl)
            fire_gather(par, sl + NBUF, sl)
            fire_scatter(par, sl, sl)

        @pl.when(g + 1 < PNG)
        def _():
            fire_idx(gbase + PGC, 1 - par, isems[1 - par])

        @pl.loop(0, TRIPS - 2)
        def _(ib):
            kb = (ib + 1) * NBUF
            for sl in range(NBUF):
                k = kb + sl
                wait_gather(sl)
                wait_scatter(sl)
                scale(sl, par, k)
                fire_gather(par, k + NBUF, sl)
                fire_scatter(par, k, sl)

        @pl.when(g + 1 < PNG)
        def _():
            wait_idx(gbase + PGC, 1 - par, isems[1 - par])

        for sl in range(NBUF):
            k = PGC - NBUF + sl
            wait_gather(sl)
            wait_scatter(sl)
            scale(sl, par, k)

            @pl.when(g + 1 < PNG)
            def _():
                fire_gather(1 - par, sl, sl)

            fire_scatter(par, k, sl)

    @pl.loop(0, PNG // 2)
    def _(go2):
        do_group(go2, 0)
        do_group(go2, 1)

    for sl in range(NBUF):
        wait_scatter(sl)

    plsc.subcore_barrier()
    for b in _node_blocks(sid):
        @pl.when(b < NBLK)
        def _():
            @pl.when(c == 0)
            def _():
                pltpu.sync_copy(acc.at[pl.ds(b * ZBLK, ZBLK)],
                                out0_hbm.at[pl.ds(b * ZBLK, ZBLK)])

            @pl.when(c == 1)
            def _():
                pltpu.sync_copy(acc.at[pl.ds(b * ZBLK, ZBLK)],
                                out1_hbm.at[pl.ds(b * ZBLK, ZBLK)])


def _prep_body(deg0_ref, deg1_ref, xr_ref, w1p_ref, dinvp_ref, h1p_ref):
    dp = lax.rsqrt(deg0_ref[...] + deg1_ref[...] + 1.0)
    dinvp_ref[...] = dp
    h = jnp.dot(xr_ref[...], w1p_ref[...], preferred_element_type=jnp.float32)
    h1p_ref[...] = h * dp


def _prep_call(deg0, deg1, xr, w1p):
    return pl.pallas_call(
        _prep_body,
        grid=(TGRID,),
        in_specs=[
            pl.BlockSpec((PB, 128), lambda i: (i, 0)),
            pl.BlockSpec((PB, 128), lambda i: (i, 0)),
            pl.BlockSpec((PB, 8 * DIN), lambda i: (i, 0)),
            pl.BlockSpec((8 * DIN, 128), lambda i: (0, 0)),
        ],
        out_specs=[
            pl.BlockSpec((PB, 128), lambda i: (i, 0)),
            pl.BlockSpec((PB, 128), lambda i: (i, 0)),
        ],
        out_shape=[
            jax.ShapeDtypeStruct((NPK, 128), jnp.float32),
            jax.ShapeDtypeStruct((NPK, 128), jnp.float32),
        ],
    )(deg0, deg1, xr, w1p)


def _mid_body(p0_ref, p1_ref, h1p_ref, dinvp_ref, b1p_ref, gp_ref):
    dp = dinvp_ref[...]
    t = p0_ref[...] + p1_ref[...] + h1p_ref[...]
    g = jnp.maximum(t * dp + b1p_ref[...], 0.0)
    gp_ref[...] = g * dp


def _mid_call(p0, p1, h1p, dinvp, b1p):
    return pl.pallas_call(
        _mid_body,
        grid=(TGRID,),
        in_specs=[
            pl.BlockSpec((PB, 128), lambda i: (i, 0)),
            pl.BlockSpec((PB, 128), lambda i: (i, 0)),
            pl.BlockSpec((PB, 128), lambda i: (i, 0)),
            pl.BlockSpec((PB, 128), lambda i: (i, 0)),
            pl.BlockSpec((1, 128), lambda i: (0, 0)),
        ],
        out_specs=pl.BlockSpec((PB, 128), lambda i: (i, 0)),
        out_shape=jax.ShapeDtypeStruct((NPK, 128), jnp.float32),
    )(p0, p1, h1p, dinvp, b1p)


def _fin_body(q0_ref, q1_ref, gp_ref, dinvp_ref, w2p_ref, b2p_ref, out_ref):
    t = (q0_ref[...] + q1_ref[...] + gp_ref[...]) * dinvp_ref[...]
    out_ref[...] = (
        jnp.dot(t, w2p_ref[...], preferred_element_type=jnp.float32)
        + b2p_ref[...]
    )


def _fin_call(q0, q1, gp, dinvp, w2p, b2p):
    return pl.pallas_call(
        _fin_body,
        grid=(TGRID,),
        in_specs=[
            pl.BlockSpec((PB, 128), lambda i: (i, 0)),
            pl.BlockSpec((PB, 128), lambda i: (i, 0)),
            pl.BlockSpec((PB, 128), lambda i: (i, 0)),
            pl.BlockSpec((PB, 128), lambda i: (i, 0)),
            pl.BlockSpec((128, 8 * DOUT), lambda i: (0, 0)),
            pl.BlockSpec((1, 8 * DOUT), lambda i: (0, 0)),
        ],
        out_specs=pl.BlockSpec((PB, 8 * DOUT), lambda i: (i, 0)),
        out_shape=jax.ShapeDtypeStruct((NPK, 8 * DOUT), jnp.float32),
    )(q0, q1, gp, dinvp, w2p, b2p)


@functools.cache
def _sc_calls():
    mesh = plsc.VectorSubcoreMesh(
        core_axis_name="c", subcore_axis_name="s", num_cores=NC, num_subcores=NS
    )
    deg = pl.kernel(
        _deg_kernel,
        out_type=[jax.ShapeDtypeStruct((N, DH), jnp.float32),
                  jax.ShapeDtypeStruct((N, DH), jnp.float32)],
        mesh=mesh,
        scratch_types=[
            pltpu.VMEM_SHARED((N,), jnp.float32),
            pltpu.VMEM((2, DGC, CHUNK), jnp.int32),
            pltpu.VMEM((2, DGC, CHUNK), jnp.float32),
            pltpu.VMEM((ZBLK,), jnp.float32),
            pltpu.VMEM((ZBLK, DH), jnp.float32),
            pltpu.SemaphoreType.DMA,
            [pltpu.SemaphoreType.DMA] * 2,
        ],
        compiler_params=pltpu.CompilerParams(use_tc_tiling_on_sc=False),
    )
    prop = pl.kernel(
        _prop_kernel,
        out_type=[jax.ShapeDtypeStruct((N, DH), jnp.float32),
                  jax.ShapeDtypeStruct((N, DH), jnp.float32)],
        mesh=mesh,
        scratch_types=[
            pltpu.VMEM_SHARED((N, DH), jnp.float32),
            pltpu.VMEM((2, PGC, CHUNK), jnp.int32),
            pltpu.VMEM((2, PGC, CHUNK), jnp.int32),
            pltpu.VMEM((2, PGC, CHUNK), jnp.float32),
            pltpu.VMEM((NBUF, CHUNK, DH), jnp.float32),
            pltpu.VMEM((NBUF, CHUNK, DH), jnp.float32),
            [pltpu.SemaphoreType.DMA] * NBUF,
            [pltpu.SemaphoreType.DMA] * NBUF,
            [pltpu.SemaphoreType.DMA] * 2,
        ],
        compiler_params=pltpu.CompilerParams(use_tc_tiling_on_sc=False),
    )
    return deg, prop


def kernel(x, edge_index, edge_weight, W1, b1, W2, b2):
    _deg_call, _prop_call = _sc_calls()
    src = edge_index[0].astype(jnp.int32).reshape(E // CHUNK, CHUNK)
    dst = edge_index[1].astype(jnp.int32).reshape(E // CHUNK, CHUNK)
    ew = edge_weight.astype(jnp.float32).reshape(E // CHUNK, CHUNK)
    zrow = jnp.zeros((ZBLK,), jnp.float32)
    zblk = jnp.zeros((ZBLK, DH), jnp.float32)
    w1p = jnp.kron(jnp.eye(8, dtype=jnp.float32), W1.astype(jnp.float32))
    w2p = jnp.kron(jnp.eye(8, dtype=jnp.float32), W2.astype(jnp.float32))
    b1p = jnp.tile(b1.astype(jnp.float32), 8).reshape(1, 128)
    b2p = jnp.tile(b2.astype(jnp.float32), 8).reshape(1, 8 * DOUT)
    xr = x.reshape(NPK, 8 * DIN)

    deg0, deg1 = _deg_call(dst, ew, zrow)
    dinvp, h1p = _prep_call(deg0.reshape(NPK, 128), deg1.reshape(NPK, 128),
                            xr, w1p)
    p0, p1 = _prop_call(h1p.reshape(N, DH), src, dst, ew, zblk)
    gp = _mid_call(p0.reshape(NPK, 128), p1.reshape(NPK, 128), h1p, dinvp, b1p)
    q0, q1 = _prop_call(gp.reshape(N, DH), src, dst, ew, zblk)
    out = _fin_call(q0.reshape(NPK, 128), q1.reshape(NPK, 128), gp, dinvp,
                    w2p, b2p)
    return out.reshape(N, DOUT)


# speedup vs baseline: 1.0173x; 1.0173x over previous
import functools

import jax
import jax.numpy as jnp
from jax import lax
from jax.experimental import pallas as pl
from jax.experimental.pallas import tpu as pltpu
from jax.experimental.pallas import tpu_sc as plsc

N = 100000
DIN = 128
DH = 16
DOUT = 40
E = 3200000

NC = 2
NS = 16
NW = NC * NS
EPW = E // NW
CHUNK = 80
NCHUNKS = EPW // CHUNK

DGC = 125
DNG = NCHUNKS // DGC
PGC = 25
PNG = NCHUNKS // PGC
NBUF = 5
ZBLK = 2000
NBLK = N // ZBLK
NPK = N * DH // 128
PB = 2048
NDB = PB * 8
TGRID = (NPK + PB - 1) // PB

def _node_blocks(s):
    out = []
    for j in range((NBLK + NS - 1) // NS):
        out.append(s + j * NS)
    return out


def _deg_kernel(dst_hbm, ew_hbm, zrow_hbm, out0_hbm, out1_hbm, acc, dstb, ewb,
                buf, bufx, ssem, isems):
    c = lax.axis_index("c")
    sid = lax.axis_index("s")
    wid = sid * NC + c
    wbase = wid * NCHUNKS

    pltpu.sync_copy(zrow_hbm, buf)
    for b in _node_blocks(sid):
        @pl.when(b < NBLK)
        def _():
            pltpu.sync_copy(buf, acc.at[pl.ds(b * ZBLK, ZBLK)])
    plsc.subcore_barrier()

    pltpu.async_copy(dst_hbm.at[pl.ds(wbase, DGC)], dstb.at[0], isems[0])
    pltpu.async_copy(ew_hbm.at[pl.ds(wbase, DGC)], ewb.at[0], isems[0])

    def do_group(go2, par):
        g = go2 * 2 + par
        gbase = wbase + g * DGC

        @pl.when(g > 0)
        def _():
            @pl.loop(0, DGC)
            def _(_k):
                pltpu.make_async_copy(
                    ewb.at[1 - par, 0], acc.at[dstb.at[1 - par, 0]], ssem
                ).wait()

        pltpu.make_async_copy(dst_hbm.at[pl.ds(gbase, DGC)], dstb.at[par],
                              isems[par]).wait()
        pltpu.make_async_copy(ew_hbm.at[pl.ds(gbase, DGC)], ewb.at[par],
                              isems[par]).wait()

        @pl.when(g + 1 < DNG)
        def _():
            pltpu.async_copy(dst_hbm.at[pl.ds(gbase + DGC, DGC)],
                             dstb.at[1 - par], isems[1 - par])
            pltpu.async_copy(ew_hbm.at[pl.ds(gbase + DGC, DGC)],
                             ewb.at[1 - par], isems[1 - par])

        @pl.loop(0, DGC)
        def _(k):
            pltpu.async_copy(ewb.at[par, k], acc.at[dstb.at[par, k]], ssem,
                             add=True)

    @pl.loop(0, DNG // 2)
    def _(go2):
        do_group(go2, 0)
        do_group(go2, 1)

    @pl.loop(0, DGC)
    def _(_k):
        pltpu.make_async_copy(ewb.at[1, 0], acc.at[dstb.at[1, 0]], ssem).wait()

    plsc.subcore_barrier()
    for b in _node_blocks(sid):
        @pl.when(b < NBLK)
        def _():
            pltpu.sync_copy(acc.at[pl.ds(b * ZBLK, ZBLK)], buf)

            @pl.loop(0, ZBLK // 16)
            def _(g):
                d16 = buf[pl.ds(g * 16, 16)]
                for t in range(16):
                    bufx[g * 16 + t, :] = jnp.broadcast_to(d16[t], (DH,))

            @pl.when(c == 0)
            def _():
                pltpu.sync_copy(bufx, out0_hbm.at[pl.ds(b * ZBLK, ZBLK)])

            @pl.when(c == 1)
            def _():
                pltpu.sync_copy(bufx, out1_hbm.at[pl.ds(b * ZBLK, ZBLK)])


def _prop_kernel(hp_hbm, src_hbm, dst_hbm, ew_hbm, zblk_hbm, out0_hbm, out1_hbm,
                 acc, srcb, dstb, ewb, rows_g, rows_s, gsems, ssems, isems):
    c = lax.axis_index("c")
    sid = lax.axis_index("s")
    wid = sid * NC + c
    wbase = wid * NCHUNKS

    for b in _node_blocks(sid):
        @pl.when(b < NBLK)
        def _():
            pltpu.sync_copy(zblk_hbm, acc.at[pl.ds(b * ZBLK, ZBLK)])
    plsc.subcore_barrier()

    def fire_idx(gbase, par, sem):
        pltpu.async_copy(src_hbm.at[pl.ds(gbase, PGC)], srcb.at[par], sem)
        pltpu.async_copy(dst_hbm.at[pl.ds(gbase, PGC)], dstb.at[par], sem)
        pltpu.async_copy(ew_hbm.at[pl.ds(gbase, PGC)], ewb.at[par], sem)

    def wait_idx(gbase, par, sem):
        pltpu.make_async_copy(src_hbm.at[pl.ds(gbase, PGC)], srcb.at[par],
                              sem).wait()
        pltpu.make_async_copy(dst_hbm.at[pl.ds(gbase, PGC)], dstb.at[par],
                              sem).wait()
        pltpu.make_async_copy(ew_hbm.at[pl.ds(gbase, PGC)], ewb.at[par],
                              sem).wait()

    def scale(sl, par, k):
        for q in range(CHUNK // 16):
            ew16 = ewb[par, k, pl.ds(q * 16, 16)]
            for t in range(16):
                i = q * 16 + t
                rows_s[sl, i, :] = rows_g[sl, i, :] * ew16[t]

    def fire_gather(par, row, sl):
        pltpu.async_copy(hp_hbm.at[srcb.at[par, row]], rows_g.at[sl],
                         gsems[sl])

    def wait_gather(sl):
        pltpu.make_async_copy(hp_hbm.at[srcb.at[0, 0]], rows_g.at[sl],
                              gsems[sl]).wait()

    def fire_scatter(par, k, sl):
        pltpu.async_copy(rows_s.at[sl], acc.at[dstb.at[par, k]], ssems[sl],
                         add=True)

    def wait_scatter(sl):
        pltpu.make_async_copy(rows_s.at[sl], acc.at[dstb.at[0, 0]],
                              ssems[sl]).wait()

    fire_idx(wbase, 0, isems[0])
    wait_idx(wbase, 0, isems[0])
    for sl in range(NBUF):
        fire_gather(0, sl, sl)

    TRIPS = PGC // NBUF

    def do_group(go2, par):
        g = go2 * 2 + par
        gbase = wbase + g * PGC

        for sl in range(NBUF):
            wait_gather(sl)

            @pl.when(g > 0)
            def _():
                wait_scatter(sl)

            scale(sl, par, sl)
            fire_gather(par, sl + NBUF, sl)
            fire_scatter(par, sl, sl)

        @pl.when(g + 1 < PNG)
        def _():
            fire_idx(gbase + PGC, 1 - par, isems[1 - par])

        @pl.loop(0, TRIPS - 2)
        def _(ib):
            kb = (ib + 1) * NBUF
            for sl in range(NBUF):
                k = kb + sl
                wait_gather(sl)
                wait_scatter(sl)
                scale(sl, par, k)
                fire_gather(par, k + NBUF, sl)
                fire_scatter(par, k, sl)

        @pl.when(g + 1 < PNG)
        def _():
            wait_idx(gbase + PGC, 1 - par, isems[1 - par])

        for sl in range(NBUF):
            k = PGC - NBUF + sl
            wait_gather(sl)
            wait_scatter(sl)
            scale(sl, par, k)

            @pl.when(g + 1 < PNG)
            def _():
                fire_gather(1 - par, sl, sl)

            fire_scatter(par, k, sl)

    @pl.loop(0, PNG // 2)
    def _(go2):
        do_group(go2, 0)
        do_group(go2, 1)

    for sl in range(NBUF):
        wait_scatter(sl)

    plsc.subcore_barrier()
    for b in _node_blocks(sid):
        @pl.when(b < NBLK)
        def _():
            @pl.when(c == 0)
            def _():
                pltpu.sync_copy(acc.at[pl.ds(b * ZBLK, ZBLK)],
                                out0_hbm.at[pl.ds(b * ZBLK, ZBLK)])

            @pl.when(c == 1)
            def _():
                pltpu.sync_copy(acc.at[pl.ds(b * ZBLK, ZBLK)],
                                out1_hbm.at[pl.ds(b * ZBLK, ZBLK)])


def _prep_body(deg0_ref, deg1_ref, xr_ref, w1p_ref, dinvp_ref, h1p_ref):
    dp = lax.rsqrt(deg0_ref[...] + deg1_ref[...] + 1.0)
    dinvp_ref[...] = dp
    h = jnp.dot(xr_ref[...], w1p_ref[...], preferred_element_type=jnp.float32)
    h1p_ref[...] = h * dp


def _prep_call(deg0, deg1, xr, w1p):
    return pl.pallas_call(
        _prep_body,
        grid=(TGRID,),
        in_specs=[
            pl.BlockSpec((PB, 128), lambda i: (i, 0)),
            pl.BlockSpec((PB, 128), lambda i: (i, 0)),
            pl.BlockSpec((PB, 8 * DIN), lambda i: (i, 0)),
            pl.BlockSpec((8 * DIN, 128), lambda i: (0, 0)),
        ],
        out_specs=[
            pl.BlockSpec((PB, 128), lambda i: (i, 0)),
            pl.BlockSpec((PB, 128), lambda i: (i, 0)),
        ],
        out_shape=[
            jax.ShapeDtypeStruct((NPK, 128), jnp.float32),
            jax.ShapeDtypeStruct((NPK, 128), jnp.float32),
        ],
    )(deg0, deg1, xr, w1p)


def _mid_body(p0_ref, p1_ref, h1p_ref, dinvp_ref, b1p_ref, gp_ref):
    dp = dinvp_ref[...]
    t = p0_ref[...] + p1_ref[...] + h1p_ref[...]
    g = jnp.maximum(t * dp + b1p_ref[...], 0.0)
    gp_ref[...] = g * dp


def _mid_call(p0, p1, h1p, dinvp, b1p):
    return pl.pallas_call(
        _mid_body,
        grid=(TGRID,),
        in_specs=[
            pl.BlockSpec((PB, 128), lambda i: (i, 0)),
            pl.BlockSpec((PB, 128), lambda i: (i, 0)),
            pl.BlockSpec((PB, 128), lambda i: (i, 0)),
            pl.BlockSpec((PB, 128), lambda i: (i, 0)),
            pl.BlockSpec((1, 128), lambda i: (0, 0)),
        ],
        out_specs=pl.BlockSpec((PB, 128), lambda i: (i, 0)),
        out_shape=jax.ShapeDtypeStruct((NPK, 128), jnp.float32),
    )(p0, p1, h1p, dinvp, b1p)


def _fin_body(q0_ref, q1_ref, gp_ref, dinvp_ref, w2p_ref, b2p_ref, out_ref):
    t = (q0_ref[...] + q1_ref[...] + gp_ref[...]) * dinvp_ref[...]
    out_ref[...] = (
        jnp.dot(t, w2p_ref[...], preferred_element_type=jnp.float32)
        + b2p_ref[...]
    )


def _fin_call(q0, q1, gp, dinvp, w2p, b2p):
    return pl.pallas_call(
        _fin_body,
        grid=(TGRID,),
        in_specs=[
            pl.BlockSpec((PB, 128), lambda i: (i, 0)),
            pl.BlockSpec((PB, 128), lambda i: (i, 0)),
            pl.BlockSpec((PB, 128), lambda i: (i, 0)),
            pl.BlockSpec((PB, 128), lambda i: (i, 0)),
            pl.BlockSpec((128, 8 * DOUT), lambda i: (0, 0)),
            pl.BlockSpec((1, 8 * DOUT), lambda i: (0, 0)),
        ],
        out_specs=pl.BlockSpec((PB, 8 * DOUT), lambda i: (i, 0)),
        out_shape=jax.ShapeDtypeStruct((NPK, 8 * DOUT), jnp.float32),
    )(q0, q1, gp, dinvp, w2p, b2p)


@functools.cache
def _sc_calls():
    mesh = plsc.VectorSubcoreMesh(
        core_axis_name="c", subcore_axis_name="s", num_cores=NC, num_subcores=NS
    )
    deg = pl.kernel(
        _deg_kernel,
        out_type=[jax.ShapeDtypeStruct((N, DH), jnp.float32),
                  jax.ShapeDtypeStruct((N, DH), jnp.float32)],
        mesh=mesh,
        scratch_types=[
            pltpu.VMEM_SHARED((N,), jnp.float32),
            pltpu.VMEM((2, DGC, CHUNK), jnp.int32),
            pltpu.VMEM((2, DGC, CHUNK), jnp.float32),
            pltpu.VMEM((ZBLK,), jnp.float32),
            pltpu.VMEM((ZBLK, DH), jnp.float32),
            pltpu.SemaphoreType.DMA,
            [pltpu.SemaphoreType.DMA] * 2,
        ],
        compiler_params=pltpu.CompilerParams(use_tc_tiling_on_sc=False),
    )
    prop = pl.kernel(
        _prop_kernel,
        out_type=[jax.ShapeDtypeStruct((N, DH), jnp.float32),
                  jax.ShapeDtypeStruct((N, DH), jnp.float32)],
        mesh=mesh,
        scratch_types=[
            pltpu.VMEM_SHARED((N, DH), jnp.float32),
            pltpu.VMEM((2, PGC, CHUNK), jnp.int32),
            pltpu.VMEM((2, PGC, CHUNK), jnp.int32),
            pltpu.VMEM((2, PGC, CHUNK), jnp.float32),
            pltpu.VMEM((NBUF, CHUNK, DH), jnp.float32),
            pltpu.VMEM((NBUF, CHUNK, DH), jnp.float32),
            [pltpu.SemaphoreType.DMA] * NBUF,
            [pltpu.SemaphoreType.DMA] * NBUF,
            [pltpu.SemaphoreType.DMA] * 2,
        ],
        compiler_params=pltpu.CompilerParams(use_tc_tiling_on_sc=False),
    )
    return deg, prop


def kernel(x, edge_index, edge_weight, W1, b1, W2, b2):
    _deg_call, _prop_call = _sc_calls()
    src = edge_index[0].astype(jnp.int32).reshape(E // CHUNK, CHUNK)
    dst = edge_index[1].astype(jnp.int32).reshape(E // CHUNK, CHUNK)
    ew = edge_weight.astype(jnp.float32).reshape(E // CHUNK, CHUNK)
    zrow = jnp.zeros((ZBLK,), jnp.float32)
    zblk = jnp.zeros((ZBLK, DH), jnp.float32)
    w1p = jnp.kron(jnp.eye(8, dtype=jnp.float32), W1.astype(jnp.float32))
    w2p = jnp.kron(jnp.eye(8, dtype=jnp.float32), W2.astype(jnp.float32))
    b1p = jnp.tile(b1.astype(jnp.float32), 8).reshape(1, 128)
    b2p = jnp.tile(b2.astype(jnp.float32), 8).reshape(1, 8 * DOUT)
    xr = x.reshape(NPK, 8 * DIN)

    deg0, deg1 = _deg_call(dst, ew, zrow)
    dinvp, h1p = _prep_call(deg0.reshape(NPK, 128), deg1.reshape(NPK, 128),
                            xr, w1p)
    p0, p1 = _prop_call(h1p.reshape(N, DH), src, dst, ew, zblk)
    gp = _mid_call(p0.reshape(NPK, 128), p1.reshape(NPK, 128), h1p, dinvp, b1p)
    q0, q1 = _prop_call(gp.reshape(N, DH), src, dst, ew, zblk)
    out = _fin_call(q0.reshape(NPK, 128), q1.reshape(NPK, 128), gp, dinvp,
                    w2p, b2p)
    return out.reshape(N, DOUT)
